# baseline (speedup 1.0000x reference)
import functools

import jax
import jax.numpy as jnp
from jax import lax
from jax.experimental import pallas as pl
from jax.experimental.pallas import tpu as pltpu
from jax.experimental.pallas import tpu_sc as plsc

_N_TOK = 8192
_N_EMB = 8192
_DIM = 32
_T_BLK = 1024
_E_CHK = 2048
_COMMIT = 0.25

_NC, _NS = 2, 16
_NW = _NC * _NS
_BPW = _N_TOK // _NW
_ICH = 128
_KCH = _BPW // _ICH


def _dist_argmin_body(esq_ref, z_ref, emb_ref, idx_ref, loss_ref):
    pid = pl.program_id(0)
    zb = z_ref[...]
    zsq = jnp.sum(zb * zb, axis=1)

    m_cmp = jnp.full((_T_BLK,), jnp.inf, jnp.float32)
    m_val = jnp.zeros((_T_BLK,), jnp.float32)
    am = jnp.zeros((_T_BLK,), jnp.int32)
    ii = lax.broadcasted_iota(jnp.int32, (_T_BLK, _E_CHK), 1)
    for c in range(_N_EMB // _E_CHK):
        eb = emb_ref[pl.ds(c * _E_CHK, _E_CHK), :]
        sb = esq_ref[0, pl.ds(c * _E_CHK, _E_CHK)]
        prod = lax.dot_general(zb, eb, (((1,), (1,)), ((), ())),
                               preferred_element_type=jnp.float32)
        d = (zsq[:, None] + sb[None, :]) - 2.0 * prod
        cm = jnp.min(d, axis=1)
        cam = jnp.min(jnp.where(d == cm[:, None], ii, _N_EMB), axis=1)
        take = cm < m_cmp
        m_val = jnp.where(take, cm, m_val)
        am = jnp.where(take, cam + c * _E_CHK, am)
        m_cmp = jnp.where(take, cm, m_cmp).astype(jnp.bfloat16).astype(
            jnp.float32)

    idx_ref[0, 0, :] = am

    @pl.when(pid == 0)
    def _():
        loss_ref[0, 0] = 0.0

    loss_ref[0, 0] += jnp.sum(m_val)


def _tc_dist_argmin(z_flat, esq, embedding):
    grid = (_N_TOK // _T_BLK,)
    return pl.pallas_call(
        _dist_argmin_body,
        grid=grid,
        in_specs=[
            pl.BlockSpec((1, _N_EMB), lambda i: (0, 0)),
            pl.BlockSpec((_T_BLK, _DIM), lambda i: (i, 0)),
            pl.BlockSpec((_N_EMB, _DIM), lambda i: (0, 0)),
        ],
        out_specs=[
            pl.BlockSpec((1, 1, _T_BLK), lambda i: (i, 0, 0)),
            pl.BlockSpec((1, 1), lambda i: (0, 0),
                         memory_space=pltpu.SMEM),
        ],
        out_shape=[
            jax.ShapeDtypeStruct((grid[0], 1, _T_BLK), jnp.int32),
            jax.ShapeDtypeStruct((1, 1), jnp.float32),
        ],
        compiler_params=pltpu.CompilerParams(
            dimension_semantics=("arbitrary",)),
    )(esq, z_flat, embedding)


def _sc_gather_body(emb_hbm, idx_hbm, zraw_hbm, out_hbm,
                    idx_v, rows_v, zrows_v, sem):
    wid = lax.axis_index("s") * _NC + lax.axis_index("c")
    base = wid * _BPW
    pltpu.sync_copy(idx_hbm.at[pl.ds(wid * _KCH, _KCH)], idx_v)
    pltpu.sync_copy(zraw_hbm.at[pl.ds(base, _BPW)], zrows_v)
    for j in range(_KCH):
        pltpu.async_copy(emb_hbm.at[idx_v.at[j]],
                         rows_v.at[pl.ds(j * _ICH, _ICH)], sem).wait()

    def row(t, carry):
        for h in range(_DIM // 16):
            q = rows_v[t, pl.ds(h * 16, 16)]
            zv = zrows_v[t, pl.ds(h * 16, 16)]
            rows_v[t, pl.ds(h * 16, 16)] = zv + (q - zv)
        return carry

    lax.fori_loop(0, _BPW, row, 0)
    pltpu.sync_copy(rows_v, out_hbm.at[pl.ds(base, _BPW)])


def _sc_gather(embedding, idx2d, zraw):
    mesh = plsc.VectorSubcoreMesh(core_axis_name="c", subcore_axis_name="s")
    k = functools.partial(
        pl.kernel,
        out_type=jax.ShapeDtypeStruct((_N_TOK, _DIM), jnp.float32),
        mesh=mesh,
        scratch_types=[
            pltpu.VMEM((_KCH, _ICH), jnp.int32),
            pltpu.VMEM((_BPW, _DIM), jnp.float32),
            pltpu.VMEM((_BPW, _DIM), jnp.float32),
            pltpu.SemaphoreType.DMA,
        ],
        compiler_params=pltpu.CompilerParams(use_tc_tiling_on_sc=False),
    )(_sc_gather_body)
    return k(embedding, idx2d, zraw)


def kernel(z, embedding):
    B, C, H, W = z.shape
    z_flat = jnp.transpose(z, (0, 2, 3, 1)).reshape(-1, C)
    esq = jnp.sum(embedding ** 2, axis=1).reshape(1, _N_EMB)

    idx3d, loss_sum = _tc_dist_argmin(z_flat, esq, embedding)

    idx2d = idx3d.reshape(_N_TOK // _ICH, _ICH)
    zraw = z.reshape(_N_TOK, _DIM)
    quantized_flat = _sc_gather(embedding, idx2d, zraw)

    quantized = quantized_flat.reshape(z.shape)
    loss = loss_sum[0, 0] * ((1.0 + _COMMIT) / (_N_TOK * _DIM))
    encoding_indices = idx3d.reshape(B, H, W)
    return (quantized, loss, encoding_indices)


# speedup vs baseline: 1.0221x; 1.0221x over previous
import functools

import jax
import jax.numpy as jnp
from jax import lax
from jax.experimental import pallas as pl
from jax.experimental.pallas import tpu as pltpu
from jax.experimental.pallas import tpu_sc as plsc

_N_TOK = 8192
_N_EMB = 8192
_DIM = 32
_T_BLK = 2048
_E_CHK = 2048
_COMMIT = 0.25

_NC, _NS = 2, 16
_NW = _NC * _NS
_BPW = _N_TOK // _NW
_ICH = 128
_KCH = _BPW // _ICH


def _dist_argmin_body(esq_ref, z_ref, emb_ref, idx_ref, loss_ref):
    pid = pl.program_id(0)
    zb = z_ref[...]
    zsq = jnp.sum(zb * zb, axis=1)

    m_cmp = jnp.full((_T_BLK,), jnp.inf, jnp.float32)
    m_val = jnp.zeros((_T_BLK,), jnp.float32)
    am = jnp.zeros((_T_BLK,), jnp.int32)
    ii = lax.broadcasted_iota(jnp.int32, (_T_BLK, _E_CHK), 1)
    for c in range(_N_EMB // _E_CHK):
        eb = emb_ref[pl.ds(c * _E_CHK, _E_CHK), :]
        sb = esq_ref[0, pl.ds(c * _E_CHK, _E_CHK)]
        prod = lax.dot_general(zb, eb, (((1,), (1,)), ((), ())),
                               preferred_element_type=jnp.float32)
        d = (zsq[:, None] + sb[None, :]) - 2.0 * prod
        cm = jnp.min(d, axis=1)
        cam = jnp.min(jnp.where(d == cm[:, None], ii, _N_EMB), axis=1)
        take = cm < m_cmp
        m_val = jnp.where(take, cm, m_val)
        am = jnp.where(take, cam + c * _E_CHK, am)
        m_cmp = jnp.where(take, cm, m_cmp).astype(jnp.bfloat16).astype(
            jnp.float32)

    idx_ref[0, 0, :] = am

    @pl.when(pid == 0)
    def _():
        loss_ref[0, 0] = 0.0

    loss_ref[0, 0] += jnp.sum(m_val)


def _tc_dist_argmin(z_flat, esq, embedding):
    grid = (_N_TOK // _T_BLK,)
    return pl.pallas_call(
        _dist_argmin_body,
        grid=grid,
        in_specs=[
            pl.BlockSpec((1, _N_EMB), lambda i: (0, 0)),
            pl.BlockSpec((_T_BLK, _DIM), lambda i: (i, 0)),
            pl.BlockSpec((_N_EMB, _DIM), lambda i: (0, 0)),
        ],
        out_specs=[
            pl.BlockSpec((1, 1, _T_BLK), lambda i: (i, 0, 0)),
            pl.BlockSpec((1, 1), lambda i: (0, 0),
                         memory_space=pltpu.SMEM),
        ],
        out_shape=[
            jax.ShapeDtypeStruct((grid[0], 1, _T_BLK), jnp.int32),
            jax.ShapeDtypeStruct((1, 1), jnp.float32),
        ],
        compiler_params=pltpu.CompilerParams(
            dimension_semantics=("arbitrary",)),
    )(esq, z_flat, embedding)


def _sc_gather_body(emb_hbm, idx_hbm, zraw_hbm, out_hbm,
                    idx_v, rows_v, zrows_v, sem):
    wid = lax.axis_index("s") * _NC + lax.axis_index("c")
    base = wid * _BPW
    pltpu.sync_copy(idx_hbm.at[pl.ds(wid * _KCH, _KCH)], idx_v)
    pltpu.sync_copy(zraw_hbm.at[pl.ds(base, _BPW)], zrows_v)
    for j in range(_KCH):
        pltpu.async_copy(emb_hbm.at[idx_v.at[j]],
                         rows_v.at[pl.ds(j * _ICH, _ICH)], sem).wait()

    def row(t, carry):
        for h in range(_DIM // 16):
            q = rows_v[t, pl.ds(h * 16, 16)]
            zv = zrows_v[t, pl.ds(h * 16, 16)]
            rows_v[t, pl.ds(h * 16, 16)] = zv + (q - zv)
        return carry

    lax.fori_loop(0, _BPW, row, 0)
    pltpu.sync_copy(rows_v, out_hbm.at[pl.ds(base, _BPW)])


def _sc_gather(embedding, idx2d, zraw):
    mesh = plsc.VectorSubcoreMesh(core_axis_name="c", subcore_axis_name="s")
    k = functools.partial(
        pl.kernel,
        out_type=jax.ShapeDtypeStruct((_N_TOK, _DIM), jnp.float32),
        mesh=mesh,
        scratch_types=[
            pltpu.VMEM((_KCH, _ICH), jnp.int32),
            pltpu.VMEM((_BPW, _DIM), jnp.float32),
            pltpu.VMEM((_BPW, _DIM), jnp.float32),
            pltpu.SemaphoreType.DMA,
        ],
        compiler_params=pltpu.CompilerParams(use_tc_tiling_on_sc=False),
    )(_sc_gather_body)
    return k(embedding, idx2d, zraw)


def kernel(z, embedding):
    B, C, H, W = z.shape
    z_flat = jnp.transpose(z, (0, 2, 3, 1)).reshape(-1, C)
    esq = jnp.sum(embedding ** 2, axis=1).reshape(1, _N_EMB)

    idx3d, loss_sum = _tc_dist_argmin(z_flat, esq, embedding)

    idx2d = idx3d.reshape(_N_TOK // _ICH, _ICH)
    zraw = z.reshape(_N_TOK, _DIM)
    quantized_flat = _sc_gather(embedding, idx2d, zraw)

    quantized = quantized_flat.reshape(z.shape)
    loss = loss_sum[0, 0] * ((1.0 + _COMMIT) / (_N_TOK * _DIM))
    encoding_indices = idx3d.reshape(B, H, W)
    return (quantized, loss, encoding_indices)
